# Initial kernel scaffold; baseline (speedup 1.0000x reference)
import jax
import jax.numpy as jnp
from jax.experimental import pallas as pl


def kernel(x, pad_seg_embed_token, token_table, seg_table):
    raise NotImplementedError("write your pallas kernel here")


# speedup vs baseline: 1.4162x; 1.4162x over previous
import functools

import numpy as np
import jax
import jax.numpy as jnp
from jax import lax
from jax.experimental import pallas as pl
from jax.experimental.pallas import tpu as pltpu
from jax.experimental.pallas import tpu_sc as plsc

VOCAB = 100000
EMBED = 768
BATCH = 32
SEQ = 512
MAX_POS = 512

NC, NS, L = 2, 16, 16
NW = NC * NS
TOK_PER_W = (BATCH * SEQ) // NW
CH = 32
NCHUNK = TOK_PER_W // CH


def _positional_encoding_np():
    pos = np.arange(MAX_POS)[:, np.newaxis]
    i = np.arange(EMBED)[np.newaxis, :]
    angle_rates = 1 / np.power(10000, 2 * (i // 2) / np.float32(EMBED))
    angle_rads = pos * angle_rates
    sines = np.sin(angle_rads[:, 0::2])
    cosines = np.cos(angle_rads[:, 1::2])
    return np.concatenate([sines, cosines], axis=-1).astype(np.float32)


_POS_ENC = _positional_encoding_np()


def _comb_body(pos_ref, seg_ref, out_ref):
    out_ref[...] = pos_ref[...][:, None, :] + seg_ref[...][None, :, :]


def _build_comb(seg_table):
    pos = jnp.asarray(_POS_ENC)
    out = pl.pallas_call(
        _comb_body,
        out_shape=jax.ShapeDtypeStruct((MAX_POS, 3, EMBED), jnp.float32),
    )(pos, seg_table)
    return out.reshape(MAX_POS * 3, EMBED)


_MESH = plsc.VectorSubcoreMesh(
    core_axis_name="c", subcore_axis_name="s", num_cores=NC, num_subcores=NS)


@functools.partial(
    pl.kernel,
    out_type=jax.ShapeDtypeStruct((BATCH * SEQ, EMBED), jnp.float32),
    mesh=_MESH,
    scratch_types=[
        pltpu.VMEM((TOK_PER_W,), jnp.int32),
        pltpu.VMEM((TOK_PER_W,), jnp.int32),
        pltpu.VMEM((CH, EMBED), jnp.float32),
        pltpu.VMEM((CH, EMBED), jnp.float32),
        pltpu.VMEM((CH, EMBED), jnp.float32),
        pltpu.VMEM((CH, EMBED), jnp.float32),
        pltpu.SemaphoreType.DMA((2,)),
        pltpu.SemaphoreType.DMA((2,)),
        pltpu.SemaphoreType.DMA((2,)),
    ],
)
def _embed_sc(tok_hbm, comb_hbm, x_hbm, seg_hbm, out_hbm,
              idx_v, idx2_v, rows0, rows1, comb0, comb1,
              gsem, csem, ssem):
    rows = (rows0, rows1)
    combv = (comb0, comb1)
    wid = lax.axis_index("s") * NC + lax.axis_index("c")
    base = wid * TOK_PER_W

    pltpu.sync_copy(x_hbm.at[pl.ds(base, TOK_PER_W)], idx_v)
    pltpu.sync_copy(seg_hbm.at[pl.ds(base, TOK_PER_W)], idx2_v)
    for i in range(TOK_PER_W // L):
        g = idx2_v[pl.ds(i * L, L)]
        idx2_v[pl.ds(i * L, L)] = g + lax.iota(jnp.int32, L) * 3 + (3 * i * L)

    def kick(c):
        b = c % 2
        dt = pltpu.async_copy(
            tok_hbm.at[idx_v.at[pl.ds(c * CH, CH)]], rows[b], gsem.at[b])
        dc = pltpu.async_copy(
            comb_hbm.at[idx2_v.at[pl.ds(c * CH, CH)]], combv[b], csem.at[b])
        return dt, dc

    pending = {0: kick(0)}
    stores = {}
    for c in range(NCHUNK):
        b = c % 2
        if c + 1 < NCHUNK:
            if c - 1 >= 0:
                stores.pop(c - 1).wait()
            pending[c + 1] = kick(c + 1)
        dt, dc = pending.pop(c)
        dt.wait()
        dc.wait()

        def add_one(t, carry, _b=b):
            r, cb = rows[_b], combv[_b]
            for k in range(EMBED // L):
                sl = pl.ds(k * L, L)
                r[t, sl] = r[t, sl] + cb[t, sl]
            return carry

        lax.fori_loop(0, CH, add_one, 0)
        stores[c] = pltpu.async_copy(
            rows[b], out_hbm.at[pl.ds(base + c * CH, CH)], ssem.at[b])
    for c in sorted(stores):
        stores.pop(c).wait()


def kernel(x, pad_seg_embed_token, token_table, seg_table):
    comb = _build_comb(seg_table)
    out = _embed_sc(token_table, comb,
                    x.reshape(-1), pad_seg_embed_token.reshape(-1))
    return out.reshape(BATCH, SEQ, EMBED)


# speedup vs baseline: 1.5411x; 1.0882x over previous
import functools

import numpy as np
import jax
import jax.numpy as jnp
from jax import lax
from jax.experimental import pallas as pl
from jax.experimental.pallas import tpu as pltpu
from jax.experimental.pallas import tpu_sc as plsc

VOCAB = 100000
EMBED = 768
BATCH = 32
SEQ = 512
MAX_POS = 512

NC, NS, L = 2, 16, 16
NW = NC * NS
TOK_PER_W = (BATCH * SEQ) // NW
CH = 32
NCHUNK = TOK_PER_W // CH


def _positional_encoding_np():
    pos = np.arange(MAX_POS)[:, np.newaxis]
    i = np.arange(EMBED)[np.newaxis, :]
    angle_rates = 1 / np.power(10000, 2 * (i // 2) / np.float32(EMBED))
    angle_rads = pos * angle_rates
    sines = np.sin(angle_rads[:, 0::2])
    cosines = np.cos(angle_rads[:, 1::2])
    return np.concatenate([sines, cosines], axis=-1).astype(np.float32)


_POS_ENC = _positional_encoding_np()


def _comb_body(pos_ref, seg_ref, out_ref):
    out_ref[...] = seg_ref[...][:, None, :] + pos_ref[...][None, :, :]


def _build_comb(seg_table):
    pos = jnp.asarray(_POS_ENC)
    out = pl.pallas_call(
        _comb_body,
        out_shape=jax.ShapeDtypeStruct((3, MAX_POS, EMBED), jnp.float32),
    )(pos, seg_table)
    return out.reshape(3 * MAX_POS, EMBED)


_MESH = plsc.VectorSubcoreMesh(
    core_axis_name="c", subcore_axis_name="s", num_cores=NC, num_subcores=NS)


@functools.partial(
    pl.kernel,
    out_type=jax.ShapeDtypeStruct((BATCH * SEQ, EMBED), jnp.float32),
    mesh=_MESH,
    scratch_types=[
        pltpu.VMEM((TOK_PER_W,), jnp.int32),
        pltpu.VMEM((TOK_PER_W,), jnp.int32),
        pltpu.VMEM((CH, EMBED), jnp.float32),
        pltpu.VMEM((CH, EMBED), jnp.float32),
        pltpu.VMEM((CH, EMBED), jnp.float32),
        pltpu.VMEM((CH, EMBED), jnp.float32),
        pltpu.SemaphoreType.DMA((2,)),
        pltpu.SemaphoreType.DMA((2,)),
        pltpu.SemaphoreType.DMA((2,)),
    ],
)
def _embed_sc(tok_hbm, comb_hbm, x_hbm, seg_hbm, out_hbm,
              idx_v, idx2_v, rows0, rows1, comb0, comb1,
              gsem, csem, ssem):
    rows = (rows0, rows1)
    combv = (comb0, comb1)
    wid = lax.axis_index("s") * NC + lax.axis_index("c")
    base = wid * TOK_PER_W

    pltpu.sync_copy(x_hbm.at[pl.ds(base, TOK_PER_W)], idx_v)
    pltpu.sync_copy(seg_hbm.at[pl.ds(base, TOK_PER_W)], idx2_v)
    for i in range(TOK_PER_W // L):
        g = idx2_v[pl.ds(i * L, L)]
        idx2_v[pl.ds(i * L, L)] = g * MAX_POS + lax.iota(jnp.int32, L) + i * L

    def kick(c):
        b = c % 2
        dt = pltpu.async_copy(
            tok_hbm.at[idx_v.at[pl.ds(c * CH, CH)]], rows[b], gsem.at[b])
        dc = pltpu.async_copy(
            comb_hbm.at[idx2_v.at[pl.ds(c * CH, CH)]], combv[b], csem.at[b])
        return dt, dc

    pending = {0: kick(0)}
    stores = {}
    for c in range(NCHUNK):
        b = c % 2
        if c + 1 < NCHUNK:
            if c - 1 >= 0:
                stores.pop(c - 1).wait()
            pending[c + 1] = kick(c + 1)
        dt, dc = pending.pop(c)
        dt.wait()
        dc.wait()

        def add_one(t, carry, _b=b):
            r, cb = rows[_b], combv[_b]
            for k in range(EMBED // L):
                sl = pl.ds(k * L, L)
                plsc.addupdate(r.at[t, sl], cb[t, sl])
            return carry

        lax.fori_loop(0, CH, add_one, 0)
        stores[c] = pltpu.async_copy(
            rows[b], out_hbm.at[pl.ds(base + c * CH, CH)], ssem.at[b])
    for c in sorted(stores):
        stores.pop(c).wait()


def kernel(x, pad_seg_embed_token, token_table, seg_table):
    comb = _build_comb(seg_table)
    out = _embed_sc(token_table, comb,
                    x.reshape(-1), pad_seg_embed_token.reshape(-1))
    return out.reshape(BATCH, SEQ, EMBED)
